# baseline (speedup 1.0000x reference)
import functools

import jax
import jax.numpy as jnp
from jax import lax
from jax.experimental import pallas as pl
from jax.experimental.pallas import tpu as pltpu
from jax.experimental.pallas import tpu_sc as plsc

VOCAB = 1000000
SEQ = 200
BATCH = 4096
EMB = 64
OUT = 16
LANES = 16
NCORES = 2
NSUB = 16
NW = NCORES * NSUB
BPW = BATCH // NW
NBUF = 4
UNROLL = 8

TCH = 512
NFULL = VOCAB // TCH
TREM = VOCAB - NFULL * TCH


def _sc_transpose(tt, remt):
    mesh = plsc.VectorSubcoreMesh(core_axis_name="c", subcore_axis_name="s")

    @functools.partial(
        pl.kernel,
        out_type=jax.ShapeDtypeStruct((VOCAB * EMB,), jnp.float32),
        mesh=mesh,
        scratch_types=[
            [pltpu.VMEM((EMB, TCH), jnp.float32) for _ in range(2)],
            [pltpu.VMEM((TCH * EMB,), jnp.float32) for _ in range(2)],
            [pltpu.SemaphoreType.DMA for _ in range(2)],
            [pltpu.SemaphoreType.DMA for _ in range(2)],
        ],
        compiler_params=pltpu.CompilerParams(
            use_tc_tiling_on_sc=True, needs_layout_passes=False),
    )
    def transpose_kernel(tt_hbm, remt_hbm, out_hbm, bufin, bufout, isems, osems):
        cid = lax.axis_index("c")
        sid = lax.axis_index("s")
        wid = sid * NCORES + cid
        lanes = lax.broadcasted_iota(jnp.int32, (LANES,), 0)
        nch = (NFULL - wid + NW - 1) // NW

        def in_copy(i, b):
            c0 = pl.multiple_of((wid + i * NW) * TCH, 128)
            return pltpu.make_async_copy(
                tt_hbm.at[:, pl.ds(c0, TCH)], bufin[b], isems[b])

        def out_copy(i, b):
            c0 = pl.multiple_of((wid + i * NW) * TCH, 128)
            return pltpu.make_async_copy(
                bufout[b], out_hbm.at[pl.ds(c0 * EMB, TCH * EMB)], osems[b])

        def transpose_chunk(src, dst, ncols):
            def col(r, _):
                rv = jnp.broadcast_to(r, (LANES,))
                for c in range(EMB // LANES):
                    v = plsc.load_gather(src, [lanes + c * LANES, rv])
                    dst[pl.ds(r * EMB + c * LANES, LANES)] = v
                return 0

            lax.fori_loop(0, ncols, col, 0)

        @pl.when(nch > 0)
        def _():
            in_copy(0, 0).start()

        @pl.when(nch > 1)
        def _():
            in_copy(1, 1).start()

        def body(i2, _):
            for k in range(2):
                i = i2 * 2 + k

                b = k

                @pl.when(i < nch)
                def _():
                    in_copy(i, b).wait()

                    @pl.when(i >= 2)
                    def _():
                        out_copy(i - 2, b).wait()

                    transpose_chunk(bufin[b], bufout[b], TCH)
                    out_copy(i, b).start()

                    @pl.when(i + 2 < nch)
                    def _():
                        in_copy(i + 2, b).start()

            return 0

        lax.fori_loop(0, (nch + 1) // 2, body, 0)

        par = (nch - 1) % 2
        for b in range(2):
            ib = jnp.where(par == b, nch - 1, nch - 2)

            @pl.when(ib >= 0)
            def _():
                out_copy(ib, b).wait()

        @pl.when(wid == NW - 1)
        def _():
            pltpu.sync_copy(remt_hbm, bufin[0].at[:, pl.ds(0, 128)])
            transpose_chunk(bufin[0], bufout[0], TREM)
            pltpu.sync_copy(bufout[0].at[pl.ds(0, TREM * EMB)],
                            out_hbm.at[pl.ds(NFULL * TCH * EMB, TREM * EMB)])

    return transpose_kernel(tt, remt)


def _sc_pool(x, table_lin):
    mesh = plsc.VectorSubcoreMesh(core_axis_name="c", subcore_axis_name="s")

    @functools.partial(
        pl.kernel,
        out_type=jax.ShapeDtypeStruct((BATCH, EMB), jnp.float32),
        mesh=mesh,
        scratch_types=[
            pltpu.VMEM((SEQ, BPW), jnp.int32),
            [pltpu.VMEM((BPW, EMB), jnp.float32) for _ in range(NBUF)],
            pltpu.VMEM((BPW, EMB), jnp.float32),
            [pltpu.SemaphoreType.DMA for _ in range(NBUF)],
        ],
        compiler_params=pltpu.CompilerParams(use_tc_tiling_on_sc=False),
    )
    def pool_kernel(x_hbm, tab_hbm, out_hbm, idx_v, rows, acc_v, sems):
        cid = lax.axis_index("c")
        sid = lax.axis_index("s")
        wid = sid * NCORES + cid
        base = wid * BPW

        pltpu.sync_copy(x_hbm.at[:, pl.ds(base, BPW)], idx_v)

        def gather(s, b):
            return pltpu.make_async_copy(tab_hbm.at[idx_v.at[s]], rows[b], sems[b])

        for b in range(NBUF):
            gather(b, b).start()

        zero = jnp.zeros((LANES,), jnp.float32)

        def zero_body(r, _):
            for c in range(EMB // LANES):
                acc_v[r, pl.ds(c * LANES, LANES)] = zero
            return 0

        lax.fori_loop(0, BPW, zero_body, 0)

        def accum(buf):
            def body(i, _):
                r0 = i * UNROLL
                for u in range(UNROLL):
                    for c in range(EMB // LANES):
                        sl = pl.ds(c * LANES, LANES)
                        plsc.addupdate(acc_v.at[r0 + u, sl], buf[r0 + u, sl])
                return 0

            lax.fori_loop(0, BPW // UNROLL, body, 0)

        def outer(g, _):
            for b in range(NBUF):
                s = g * NBUF + b
                gather(s, b).wait()
                accum(rows[b])
                ns = s + NBUF

                @pl.when(ns < SEQ)
                def _():
                    gather(ns, b).start()

            return 0

        lax.fori_loop(0, SEQ // NBUF, outer, 0)

        inv = jnp.float32(1.0 / SEQ)

        def scale_body(r, _):
            for c in range(EMB // LANES):
                sl = pl.ds(c * LANES, LANES)
                acc_v[r, sl] = acc_v[r, sl] * inv
            return 0

        lax.fori_loop(0, BPW, scale_body, 0)
        pltpu.sync_copy(acc_v, out_hbm.at[pl.ds(base, BPW), :])

    return pool_kernel(x, table_lin)


def _tc_head(pooled, W, b2d):
    blk = 512

    def head_kernel(p_ref, w_ref, b_ref, o_ref):
        logits = jnp.dot(p_ref[...], w_ref[...],
                         preferred_element_type=jnp.float32) + b_ref[...]
        m = jnp.max(logits, axis=1, keepdims=True)
        z = logits - m
        lse = jnp.log(jnp.sum(jnp.exp(z), axis=1, keepdims=True))
        o_ref[...] = z - lse

    return pl.pallas_call(
        head_kernel,
        out_shape=jax.ShapeDtypeStruct((BATCH, OUT), jnp.float32),
        grid=(BATCH // blk,),
        in_specs=[
            pl.BlockSpec((blk, EMB), lambda i: (i, 0)),
            pl.BlockSpec((EMB, OUT), lambda i: (0, 0)),
            pl.BlockSpec((1, OUT), lambda i: (0, 0)),
        ],
        out_specs=pl.BlockSpec((blk, OUT), lambda i: (i, 0)),
    )(pooled, W, b2d)


def kernel(x, table, W, b):
    tt = table.T
    remt = jnp.pad(table[VOCAB - TREM:].T, ((0, 0), (0, 128 - TREM)))
    lin = _sc_transpose(tt, remt)
    pooled = _sc_pool(x, lin.reshape(VOCAB, EMB))
    return _tc_head(pooled, W, b.reshape(1, OUT))


# speedup vs baseline: 3.3734x; 3.3734x over previous
import functools

import jax
import jax.numpy as jnp
from jax import lax
from jax.experimental import pallas as pl
from jax.experimental.pallas import tpu as pltpu
from jax.experimental.pallas import tpu_sc as plsc

VOCAB = 1000000
SEQ = 200
BATCH = 4096
EMB = 64
OUT = 16
LANES = 16
NCORES = 2
NSUB = 16
NW = NCORES * NSUB
BPW = BATCH // NW
NBUF = 4
UNROLL = 8

SPLIT = 1 << 19
TBLK = 2048
TGRID = SPLIT // TBLK


def _tc_transpose(tt):
    def body(a_ref, b_ref, o_ref):
        o_ref[...] = jnp.concatenate([a_ref[...].T, b_ref[...].T], axis=1)

    return pl.pallas_call(
        body,
        out_shape=jax.ShapeDtypeStruct((SPLIT, 2 * EMB), jnp.float32),
        grid=(TGRID,),
        in_specs=[
            pl.BlockSpec((EMB, TBLK), lambda i: (0, i)),
            pl.BlockSpec((EMB, TBLK),
                         lambda i: (0, jnp.minimum(TGRID + i, VOCAB // TBLK))),
        ],
        out_specs=pl.BlockSpec((TBLK, 2 * EMB), lambda i: (i, 0)),
    )(tt, tt)


def _sc_pool(x, table_lin):
    mesh = plsc.VectorSubcoreMesh(core_axis_name="c", subcore_axis_name="s")

    @functools.partial(
        pl.kernel,
        out_type=jax.ShapeDtypeStruct((BATCH, EMB), jnp.float32),
        mesh=mesh,
        scratch_types=[
            pltpu.VMEM((SEQ, BPW), jnp.int32),
            [pltpu.VMEM((BPW, EMB), jnp.float32) for _ in range(NBUF)],
            pltpu.VMEM((BPW, EMB), jnp.float32),
            [pltpu.SemaphoreType.DMA for _ in range(NBUF)],
        ],
        compiler_params=pltpu.CompilerParams(use_tc_tiling_on_sc=False),
    )
    def pool_kernel(x_hbm, tab_hbm, out_hbm, idx_v, rows, acc_v, sems):
        cid = lax.axis_index("c")
        sid = lax.axis_index("s")
        wid = sid * NCORES + cid
        base = wid * BPW

        pltpu.sync_copy(x_hbm.at[:, pl.ds(base, BPW)], idx_v)

        def remap_body(k, _):
            sl = pl.ds(k * LANES, LANES)
            i0 = idx_v[k // (BPW // LANES), pl.ds((k % (BPW // LANES)) * LANES, LANES)]
            h = lax.shift_right_logical(i0, 19)
            idx_v[k // (BPW // LANES), pl.ds((k % (BPW // LANES)) * LANES, LANES)] = (
                i0 * 2 - h * (2 * SPLIT - 1))
            return 0

        lax.fori_loop(0, SEQ * BPW // LANES, remap_body, 0)

        def gather(s, b):
            return pltpu.make_async_copy(tab_hbm.at[idx_v.at[s]], rows[b], sems[b])

        for b in range(NBUF):
            gather(b, b).start()

        zero = jnp.zeros((LANES,), jnp.float32)

        def zero_body(r, _):
            for c in range(EMB // LANES):
                acc_v[r, pl.ds(c * LANES, LANES)] = zero
            return 0

        lax.fori_loop(0, BPW, zero_body, 0)

        def accum(buf):
            def body(i, _):
                r0 = i * UNROLL
                for u in range(UNROLL):
                    for c in range(EMB // LANES):
                        sl = pl.ds(c * LANES, LANES)
                        plsc.addupdate(acc_v.at[r0 + u, sl], buf[r0 + u, sl])
                return 0

            lax.fori_loop(0, BPW // UNROLL, body, 0)

        def outer(g, _):
            for b in range(NBUF):
                s = g * NBUF + b
                gather(s, b).wait()
                accum(rows[b])
                ns = s + NBUF

                @pl.when(ns < SEQ)
                def _():
                    gather(ns, b).start()

            return 0

        lax.fori_loop(0, SEQ // NBUF, outer, 0)

        inv = jnp.float32(1.0 / SEQ)

        def scale_body(r, _):
            for c in range(EMB // LANES):
                sl = pl.ds(c * LANES, LANES)
                acc_v[r, sl] = acc_v[r, sl] * inv
            return 0

        lax.fori_loop(0, BPW, scale_body, 0)
        pltpu.sync_copy(acc_v, out_hbm.at[pl.ds(base, BPW), :])

    return pool_kernel(x, table_lin)


def _tc_head(pooled, W, b2d):
    blk = 512

    def head_kernel(p_ref, w_ref, b_ref, o_ref):
        logits = jnp.dot(p_ref[...], w_ref[...],
                         preferred_element_type=jnp.float32) + b_ref[...]
        m = jnp.max(logits, axis=1, keepdims=True)
        z = logits - m
        lse = jnp.log(jnp.sum(jnp.exp(z), axis=1, keepdims=True))
        o_ref[...] = z - lse

    return pl.pallas_call(
        head_kernel,
        out_shape=jax.ShapeDtypeStruct((BATCH, OUT), jnp.float32),
        grid=(BATCH // blk,),
        in_specs=[
            pl.BlockSpec((blk, EMB), lambda i: (i, 0)),
            pl.BlockSpec((EMB, OUT), lambda i: (0, 0)),
            pl.BlockSpec((1, OUT), lambda i: (0, 0)),
        ],
        out_specs=pl.BlockSpec((blk, OUT), lambda i: (i, 0)),
    )(pooled, W, b2d)


def kernel(x, table, W, b):
    pairs = _tc_transpose(table.T)
    pooled = _sc_pool(x, pairs.reshape(2 * SPLIT, EMB))
    return _tc_head(pooled, W, b.reshape(1, OUT))


# speedup vs baseline: 3.9402x; 1.1680x over previous
import functools

import jax
import jax.numpy as jnp
from jax import lax
from jax.experimental import pallas as pl
from jax.experimental.pallas import tpu as pltpu
from jax.experimental.pallas import tpu_sc as plsc

VOCAB = 1000000
SEQ = 200
BATCH = 4096
EMB = 64
OUT = 16
LANES = 16
NCORES = 2
NSUB = 16
NW = NCORES * NSUB
BPW = BATCH // NW
NBUF = 4
UNROLL = 8

SPLIT = 1 << 19
TBLK = 4096
TGRID = SPLIT // TBLK


def _tc_transpose(tt):
    def body(a_ref, b_ref, o_ref):
        o_ref[:, 0:EMB] = a_ref[...].T
        o_ref[:, EMB:2 * EMB] = b_ref[...].T

    return pl.pallas_call(
        body,
        out_shape=jax.ShapeDtypeStruct((SPLIT, 2 * EMB), jnp.float32),
        grid=(TGRID,),
        in_specs=[
            pl.BlockSpec((EMB, TBLK), lambda i: (0, i)),
            pl.BlockSpec((EMB, TBLK),
                         lambda i: (0, jnp.minimum(TGRID + i, VOCAB // TBLK))),
        ],
        out_specs=pl.BlockSpec((TBLK, 2 * EMB), lambda i: (i, 0)),
    )(tt, tt)


def _sc_pool(x, table_lin):
    mesh = plsc.VectorSubcoreMesh(core_axis_name="c", subcore_axis_name="s")

    @functools.partial(
        pl.kernel,
        out_type=jax.ShapeDtypeStruct((BATCH, EMB), jnp.float32),
        mesh=mesh,
        scratch_types=[
            pltpu.VMEM((SEQ, BPW), jnp.int32),
            [pltpu.VMEM((BPW, EMB), jnp.float32) for _ in range(NBUF)],
            pltpu.VMEM((BPW, EMB), jnp.float32),
            [pltpu.SemaphoreType.DMA for _ in range(NBUF)],
        ],
        compiler_params=pltpu.CompilerParams(use_tc_tiling_on_sc=False),
    )
    def pool_kernel(x_hbm, tab_hbm, out_hbm, idx_v, rows, acc_v, sems):
        cid = lax.axis_index("c")
        sid = lax.axis_index("s")
        wid = sid * NCORES + cid
        base = wid * BPW

        pltpu.sync_copy(x_hbm.at[:, pl.ds(base, BPW)], idx_v)

        def remap_body(k, _):
            sl = pl.ds(k * LANES, LANES)
            i0 = idx_v[k // (BPW // LANES), pl.ds((k % (BPW // LANES)) * LANES, LANES)]
            h = lax.shift_right_logical(i0, 19)
            idx_v[k // (BPW // LANES), pl.ds((k % (BPW // LANES)) * LANES, LANES)] = (
                i0 * 2 - h * (2 * SPLIT - 1))
            return 0

        lax.fori_loop(0, SEQ * BPW // LANES, remap_body, 0)

        def gather(s, b):
            return pltpu.make_async_copy(tab_hbm.at[idx_v.at[s]], rows[b], sems[b])

        for b in range(NBUF):
            gather(b, b).start()

        zero = jnp.zeros((LANES,), jnp.float32)

        def zero_body(r, _):
            for c in range(EMB // LANES):
                acc_v[r, pl.ds(c * LANES, LANES)] = zero
            return 0

        lax.fori_loop(0, BPW, zero_body, 0)

        def accum(buf):
            def body(i, _):
                r0 = i * UNROLL
                for u in range(UNROLL):
                    for c in range(EMB // LANES):
                        sl = pl.ds(c * LANES, LANES)
                        plsc.addupdate(acc_v.at[r0 + u, sl], buf[r0 + u, sl])
                return 0

            lax.fori_loop(0, BPW // UNROLL, body, 0)

        def outer(g, _):
            for b in range(NBUF):
                s = g * NBUF + b
                gather(s, b).wait()
                accum(rows[b])
                ns = s + NBUF

                @pl.when(ns < SEQ)
                def _():
                    gather(ns, b).start()

            return 0

        lax.fori_loop(0, SEQ // NBUF, outer, 0)

        inv = jnp.float32(1.0 / SEQ)

        def scale_body(r, _):
            for c in range(EMB // LANES):
                sl = pl.ds(c * LANES, LANES)
                acc_v[r, sl] = acc_v[r, sl] * inv
            return 0

        lax.fori_loop(0, BPW, scale_body, 0)
        pltpu.sync_copy(acc_v, out_hbm.at[pl.ds(base, BPW), :])

    return pool_kernel(x, table_lin)


def _tc_head(pooled, W, b2d):
    blk = 512

    def head_kernel(p_ref, w_ref, b_ref, o_ref):
        logits = jnp.dot(p_ref[...], w_ref[...],
                         preferred_element_type=jnp.float32) + b_ref[...]
        m = jnp.max(logits, axis=1, keepdims=True)
        z = logits - m
        lse = jnp.log(jnp.sum(jnp.exp(z), axis=1, keepdims=True))
        o_ref[...] = z - lse

    return pl.pallas_call(
        head_kernel,
        out_shape=jax.ShapeDtypeStruct((BATCH, OUT), jnp.float32),
        grid=(BATCH // blk,),
        in_specs=[
            pl.BlockSpec((blk, EMB), lambda i: (i, 0)),
            pl.BlockSpec((EMB, OUT), lambda i: (0, 0)),
            pl.BlockSpec((1, OUT), lambda i: (0, 0)),
        ],
        out_specs=pl.BlockSpec((blk, OUT), lambda i: (i, 0)),
    )(pooled, W, b2d)


def kernel(x, table, W, b):
    pairs = _tc_transpose(table.T)
    pooled = _sc_pool(x, pairs.reshape(2 * SPLIT, EMB))
    return _tc_head(pooled, W, b.reshape(1, OUT))
